# baseline (speedup 1.0000x reference)
import functools

import jax
import jax.numpy as jnp
from jax import lax
from jax.experimental import pallas as pl
from jax.experimental.pallas import tpu as pltpu
from jax.experimental.pallas import tpu_sc as plsc

B = 4096
S = 32
H = 128
L = 16

NC = 2
NS = 16
NW = NC * NS
RPW = B // NW
G = 128
CW = RPW * S // G
TPW = RPW * S
HC = H // L
NBUF = 6
AHEAD = 4
RING = NBUF * G


@functools.partial(
    pl.kernel,
    out_type=jax.ShapeDtypeStruct((B, H), jnp.float32),
    mesh=plsc.VectorSubcoreMesh(core_axis_name="c", subcore_axis_name="s"),
    compiler_params=pltpu.CompilerParams(
        needs_layout_passes=False, use_tc_tiling_on_sc=False),
    scratch_types=[
        pltpu.VMEM((RPW, S), jnp.int32),
        pltpu.VMEM((TPW,), jnp.int32),
        pltpu.VMEM((RPW, S), jnp.int32),
        pltpu.VMEM((RING, H), jnp.float32),
        pltpu.VMEM((RPW, H), jnp.float32),
        pltpu.VMEM((RPW, L), jnp.int32),
        pltpu.VMEM((RPW, L), jnp.float32),
        pltpu.SemaphoreType.DMA((NBUF,)),
        pltpu.SemaphoreType.DMA,
    ],
)
def _pool_kernel(table_hbm, ids_hbm, mask_hbm, out_hbm,
                 ids_v, packed_v, mask_v, rows_v, out_v, enx_v, invx_v,
                 sems, osem):
    w = lax.axis_index("s") * NC + lax.axis_index("c")

    pltpu.sync_copy(ids_hbm.at[pl.ds(w * RPW, RPW)], ids_v)
    pltpu.sync_copy(mask_hbm.at[pl.ds(w * RPW, RPW)], mask_v)

    def dcopy(c):
        boff = (c - (c // NBUF) * NBUF) * G
        return pltpu.make_async_copy(
            table_hbm.at[packed_v.at[pl.ds(c * G, G)]],
            rows_v.at[pl.ds(boff, G)],
            sems.at[c - (c // NBUF) * NBUF])

    def pack_row(g, carry):
        off, doff = carry

        def chunk(j, c_):
            off_, doff_ = c_
            mv = mask_v[g, pl.ds(j * L, L)]
            keep = mv != 0
            incl = plsc.cumsum(mv)
            dincl = plsc.cumsum(1 - mv)
            ids_c = ids_v[g, pl.ds(j * L, L)]
            plsc.store_scatter(packed_v, [off_ + incl - 1], ids_c, mask=keep)
            plsc.store_scatter(
                packed_v, [(TPW - 1 - doff_) - (dincl - 1)], ids_c,
                mask=jnp.logical_not(keep))
            return (off_ + incl[L - 1], doff_ + dincl[L - 1])

        off2, doff2 = lax.fori_loop(0, 2, chunk, (off, doff))
        enx_v[g] = jnp.full((L,), 0, jnp.int32) + off2
        cntv = jnp.full((L,), 0.0, jnp.float32) + (off2 - off).astype(jnp.float32)
        invx_v[g] = 1.0 / jnp.maximum(cntv, 1.0)

        cc = off // G

        @pl.when((off2 // G > cc) & (cc < AHEAD))
        def _():
            dcopy(cc).start()

        return (off2, doff2)

    kept, _ = lax.fori_loop(0, RPW, pack_row, (jnp.int32(0), jnp.int32(0)))
    nch = (kept + (G - 1)) // G
    nfull = kept // G

    @pl.when((nfull < AHEAD) & (nfull < nch))
    def _():
        dcopy(nfull).start()

    def row_loop_body(carry):
        g, st, en = carry

        def s_body(p, acc):
            pidx = acc[HC]
            new = tuple(
                acc[h] + rows_v[pidx, pl.ds(h * L, L)] for h in range(HC))
            pidx1 = pidx + 1
            pidx1 = jnp.where(pidx1 == RING, 0, pidx1)
            return (*new, pidx1)

        pidx0 = st - (st // RING) * RING
        res = lax.fori_loop(
            st, en, s_body,
            tuple(jnp.zeros((L,), jnp.float32) for _ in range(HC)) + (pidx0,))
        invv = invx_v[g]
        for h in range(HC):
            out_v[g, pl.ds(h * L, L)] = res[h] * invv

        @pl.when((g & 31) == 31)
        def _():
            go = g - 31
            pltpu.make_async_copy(
                out_v.at[pl.ds(go, 32)],
                out_hbm.at[pl.ds(w * RPW + go, 32)], osem).start()

        en_next = enx_v[jnp.minimum(g + 1, RPW - 1)][0]
        return (g + 1, en, en_next)

    def process_rows(carry, limit):
        def cond(carry):
            g, st, en = carry
            return (g < RPW) & (en <= limit)

        return lax.while_loop(cond, row_loop_body, carry)

    def step(c, carry):
        @pl.when(c < nch)
        def _():
            dcopy(c).wait()

        @pl.when(c + AHEAD < nch)
        def _():
            dcopy(c + AHEAD).start()
        return process_rows(carry, (c + 1) * G)

    carry = (jnp.int32(0), jnp.int32(0), enx_v[0][0])
    lax.fori_loop(0, jnp.maximum(nch, 1), step, carry)

    for k in range(RPW // 32):
        pltpu.make_async_copy(
            out_v.at[pl.ds(k * 32, 32)],
            out_hbm.at[pl.ds(w * RPW + k * 32, 32)], osem).wait()


def kernel(ids, mask, embed_table):
    return _pool_kernel(embed_table, ids, mask)


# speedup vs baseline: 1.0051x; 1.0051x over previous
import functools

import jax
import jax.numpy as jnp
from jax import lax
from jax.experimental import pallas as pl
from jax.experimental.pallas import tpu as pltpu
from jax.experimental.pallas import tpu_sc as plsc

B = 4096
S = 32
H = 128
L = 16

NC = 2
NS = 16
NW = NC * NS
RPW = B // NW
G = 128
CW = RPW * S // G
TPW = RPW * S
HC = H // L
NBUF = 6
AHEAD = 4
RING = NBUF * G


def _build_kernel():
  decorate = functools.partial(
    pl.kernel,
    out_type=jax.ShapeDtypeStruct((B, H), jnp.float32),
    mesh=plsc.VectorSubcoreMesh(core_axis_name="c", subcore_axis_name="s"),
    compiler_params=pltpu.CompilerParams(
        needs_layout_passes=False, use_tc_tiling_on_sc=False),
    scratch_types=[
        pltpu.VMEM((RPW, S), jnp.int32),
        pltpu.VMEM((TPW,), jnp.int32),
        pltpu.VMEM((RPW, S), jnp.int32),
        pltpu.VMEM((RING, H), jnp.float32),
        pltpu.VMEM((RPW, H), jnp.float32),
        pltpu.VMEM((RPW, L), jnp.int32),
        pltpu.VMEM((RPW, L), jnp.float32),
        pltpu.SemaphoreType.DMA((NBUF,)),
        pltpu.SemaphoreType.DMA,
    ],
  )

  @decorate
  def _pool_kernel(table_hbm, ids_hbm, mask_hbm, out_hbm,
                 ids_v, packed_v, mask_v, rows_v, out_v, enx_v, invx_v,
                 sems, osem):
    w = lax.axis_index("s") * NC + lax.axis_index("c")

    pltpu.sync_copy(ids_hbm.at[pl.ds(w * RPW, RPW)], ids_v)
    pltpu.sync_copy(mask_hbm.at[pl.ds(w * RPW, RPW)], mask_v)

    def dcopy(c):
        boff = (c - (c // NBUF) * NBUF) * G
        return pltpu.make_async_copy(
            table_hbm.at[packed_v.at[pl.ds(c * G, G)]],
            rows_v.at[pl.ds(boff, G)],
            sems.at[c - (c // NBUF) * NBUF])

    def pack_row(g, carry):
        off, doff = carry

        def chunk(j, c_):
            off_, doff_ = c_
            mv = mask_v[g, pl.ds(j * L, L)]
            keep = mv != 0
            incl = plsc.cumsum(mv)
            dincl = plsc.cumsum(1 - mv)
            ids_c = ids_v[g, pl.ds(j * L, L)]
            plsc.store_scatter(packed_v, [off_ + incl - 1], ids_c, mask=keep)
            plsc.store_scatter(
                packed_v, [(TPW - 1 - doff_) - (dincl - 1)], ids_c,
                mask=jnp.logical_not(keep))
            return (off_ + incl[L - 1], doff_ + dincl[L - 1])

        off2, doff2 = lax.fori_loop(0, 2, chunk, (off, doff))
        enx_v[g] = jnp.full((L,), 0, jnp.int32) + off2
        cntv = jnp.full((L,), 0.0, jnp.float32) + (off2 - off).astype(jnp.float32)
        invx_v[g] = 1.0 / jnp.maximum(cntv, 1.0)

        cc = off // G

        @pl.when((off2 // G > cc) & (cc < AHEAD))
        def _():
            dcopy(cc).start()

        return (off2, doff2)

    kept, _ = lax.fori_loop(0, RPW, pack_row, (jnp.int32(0), jnp.int32(0)))
    nch = (kept + (G - 1)) // G
    nfull = kept // G

    @pl.when((nfull < AHEAD) & (nfull < nch))
    def _():
        dcopy(nfull).start()

    def row_loop_body(carry):
        g, st, en = carry

        def s_body(p, acc):
            pidx = acc[HC]
            new = tuple(
                acc[h] + rows_v[pidx, pl.ds(h * L, L)] for h in range(HC))
            pidx1 = pidx + 1
            pidx1 = jnp.where(pidx1 == RING, 0, pidx1)
            return (*new, pidx1)

        pidx0 = st - (st // RING) * RING
        res = lax.fori_loop(
            st, en, s_body,
            tuple(jnp.zeros((L,), jnp.float32) for _ in range(HC)) + (pidx0,))
        invv = invx_v[g]
        for h in range(HC):
            out_v[g, pl.ds(h * L, L)] = res[h] * invv

        @pl.when((g & 31) == 31)
        def _():
            go = g - 31
            pltpu.make_async_copy(
                out_v.at[pl.ds(go, 32)],
                out_hbm.at[pl.ds(w * RPW + go, 32)], osem).start()

        en_next = enx_v[jnp.minimum(g + 1, RPW - 1)][0]
        return (g + 1, en, en_next)

    def process_rows(carry, limit):
        def cond(carry):
            g, st, en = carry
            return (g < RPW) & (en <= limit)

        return lax.while_loop(cond, row_loop_body, carry)

    def step(c, carry):
        @pl.when(c < nch)
        def _():
            dcopy(c).wait()

        @pl.when(c + AHEAD < nch)
        def _():
            dcopy(c + AHEAD).start()
        return process_rows(carry, (c + 1) * G)

    carry = (jnp.int32(0), jnp.int32(0), enx_v[0][0])
    lax.fori_loop(0, jnp.maximum(nch, 1), step, carry)

    for k in range(RPW // 32):
        pltpu.make_async_copy(
            out_v.at[pl.ds(k * 32, 32)],
            out_hbm.at[pl.ds(w * RPW + k * 32, 32)], osem).wait()


  return _pool_kernel


@functools.lru_cache(maxsize=1)
def _get_kernel():
    return _build_kernel()


def kernel(ids, mask, embed_table):
    return _get_kernel()(embed_table, ids, mask)
